# baseline (speedup 1.0000x reference)
import functools

import jax
import jax.numpy as jnp
from jax import lax
from jax.experimental import pallas as pl
from jax.experimental.pallas import tpu as pltpu
from jax.experimental.pallas import tpu_sc as plsc

N = 1152
N2 = N * N
E_RAW = 19584
E_LOOPS = E_RAW + N
E_PAD = 24576
EDGES_PER_W = E_PAD // 16
CHUNKS_PER_W = EDGES_PER_W // 128
HALF = N // 2
HALF_WORDS = HALF * N
SUB_ROWS = HALF // 16
SUB_WORDS = SUB_ROWS * N
C_SH_LEN = HALF_WORDS + 128
BF = jnp.bfloat16


def _count_body(src_hbm, dst_hbm, zeros_hbm, out_hbm,
                src_v, dst_v, idx_v, val_v, c_sh, out_sem):
    cid = lax.axis_index("c")
    sid = lax.axis_index("s")
    lo = cid * HALF

    z = pltpu.async_copy(zeros_hbm, c_sh.at[pl.ds(sid * SUB_WORDS, SUB_WORDS)],
                         out_sem)
    base = pl.multiple_of(sid * EDGES_PER_W, EDGES_PER_W)
    pltpu.sync_copy(src_hbm.at[pl.ds(base, EDGES_PER_W)], src_v)
    pltpu.sync_copy(dst_hbm.at[pl.ds(base, EDGES_PER_W)], dst_v)

    for j in range(EDGES_PER_W // 16):
        d = dst_v[pl.ds(j * 16, 16)] - lo
        s = src_v[pl.ds(j * 16, 16)]
        flat = d * N + s
        ok = (d >= 0) & (d < HALF)
        idx_v[j // 8, pl.ds((j % 8) * 16, 16)] = jnp.where(ok, flat,
                                                           HALF_WORDS)
        val_v[j // 8, pl.ds((j % 8) * 16, 16)] = jnp.full((16,), 1.0,
                                                          jnp.float32)
    z.wait()
    plsc.subcore_barrier()

    for j in range(CHUNKS_PER_W):
        pltpu.sync_copy(val_v.at[j], c_sh.at[idx_v.at[j]], add=True)

    plsc.subcore_barrier()

    descs = []
    for r in range(SUB_ROWS):
        row = sid * SUB_ROWS + r
        descs.append(pltpu.async_copy(
            c_sh.at[pl.ds(row * N, N)], out_hbm.at[cid * HALF + row],
            out_sem))
    for d_ in descs:
        d_.wait()


def _build_count(src_pad, dst_pad, zeros_row):
    k = pl.kernel(
        _count_body,
        out_type=jax.ShapeDtypeStruct((N, N), jnp.float32),
        mesh=plsc.VectorSubcoreMesh(core_axis_name="c", subcore_axis_name="s"),
        scratch_types=[
            pltpu.VMEM((EDGES_PER_W,), jnp.int32),
            pltpu.VMEM((EDGES_PER_W,), jnp.int32),
            pltpu.VMEM((CHUNKS_PER_W, 128), jnp.int32),
            pltpu.VMEM((CHUNKS_PER_W, 128), jnp.float32),
            pltpu.VMEM_SHARED((C_SH_LEN,), jnp.float32),
            pltpu.SemaphoreType.DMA,
        ],
    )
    return k(src_pad, dst_pad, zeros_row)


def _attention_matrix(csum, asrc_row, adst_col):
    shift = jax.nn.leaky_relu(jnp.max(asrc_row) + jnp.max(adst_col), 0.2)
    logit = jax.nn.leaky_relu(asrc_row + adst_col, 0.2)
    a = csum * jnp.exp(logit - shift)
    denom = jnp.sum(a, axis=1, keepdims=True) + 1e-16
    return a, denom


def _l1_body(x_ref, w_ref, as_ref, ad_ref, c_ref, b_ref, out_ref):
    w1 = w_ref[...]
    x = x_ref[...]
    wv_s = jnp.dot(w1, as_ref[0], preferred_element_type=jnp.float32)
    wv_d = jnp.dot(w1, ad_ref[0], preferred_element_type=jnp.float32)
    asrc_row = lax.dot_general(wv_s, x, (((0,), (1,)), ((), ())),
                               preferred_element_type=jnp.float32)
    adst_col = jnp.dot(x, wv_d, preferred_element_type=jnp.float32)
    a, denom = _attention_matrix(c_ref[...], asrc_row, adst_col)
    ax = jnp.dot(a.astype(BF), x.astype(BF),
                 preferred_element_type=jnp.float32) * (1.0 / denom)
    h1k = jnp.dot(ax.astype(BF), w1.astype(BF),
                  preferred_element_type=jnp.float32) + b_ref[0]
    out_ref[...] = jax.nn.leaky_relu(h1k, 0.01).astype(BF)


def _layer1(x, W1, as1c, ad1c, C, b1r, heads, och):
    d_in = x.shape[1]
    return pl.pallas_call(
        _l1_body,
        grid=(heads,),
        in_specs=[
            pl.BlockSpec((N, d_in), lambda k: (0, 0)),
            pl.BlockSpec((d_in, och), lambda k: (0, k)),
            pl.BlockSpec((1, och, 1), lambda k: (k, 0, 0)),
            pl.BlockSpec((1, och, 1), lambda k: (k, 0, 0)),
            pl.BlockSpec((N, N), lambda k: (0, 0)),
            pl.BlockSpec((1, 1, och), lambda k: (k, 0, 0)),
        ],
        out_specs=pl.BlockSpec((N, och), lambda k: (0, k)),
        out_shape=jax.ShapeDtypeStruct((N, heads * och), BF),
    )(x, W1, as1c, ad1c, C, b1r)


def _mm2_body(h1_ref, w_ref, as_ref, ad_ref, out_ref, asrc_ref, adst_ref,
              acc_ref, *, kk_steps):
    kk = pl.program_id(0)
    kh = pl.program_id(1)
    part = jnp.dot(h1_ref[...], w_ref[...].astype(BF),
                   preferred_element_type=jnp.float32)

    @pl.when(kk == 0)
    def _init():
        acc_ref[kh] = part

    @pl.when(kk > 0)
    def _acc():
        acc_ref[kh] += part

    @pl.when(kk == kk_steps - 1)
    def _finish():
        hk = acc_ref[kh]
        out_ref[kh] = hk.astype(BF)
        asrc_ref[kh] = jnp.sum(hk * as_ref[kh], axis=1, keepdims=True)
        adst_ref[kh] = jnp.sum(hk * ad_ref[kh], axis=1, keepdims=True)


def _mm2(h1, W2, as2r, ad2r, heads, och, kblk):
    kin = h1.shape[1]
    kk_steps = kin // kblk
    return pl.pallas_call(
        functools.partial(_mm2_body, kk_steps=kk_steps),
        grid=(kk_steps, heads),
        in_specs=[
            pl.BlockSpec((N, kblk), lambda kk, kh: (0, kk)),
            pl.BlockSpec((kblk, och), lambda kk, kh: (kk, kh)),
            pl.BlockSpec((heads, 1, och), lambda kk, kh: (0, 0, 0)),
            pl.BlockSpec((heads, 1, och), lambda kk, kh: (0, 0, 0)),
        ],
        out_specs=[
            pl.BlockSpec((heads, N, och), lambda kk, kh: (0, 0, 0)),
            pl.BlockSpec((heads, N, 1), lambda kk, kh: (0, 0, 0)),
            pl.BlockSpec((heads, N, 1), lambda kk, kh: (0, 0, 0)),
        ],
        out_shape=[
            jax.ShapeDtypeStruct((heads, N, och), BF),
            jax.ShapeDtypeStruct((heads, N, 1), jnp.float32),
            jax.ShapeDtypeStruct((heads, N, 1), jnp.float32),
        ],
        scratch_shapes=[pltpu.VMEM((heads, N, och), jnp.float32)],
    )(h1, W2, as2r, ad2r)


def _agg2_body(c_ref, asrc_ref, adst_ref, h_ref, b2_ref, wn_ref, bn_ref,
               wgt_ref, wgb_ref, bg_ref, gf_ref, node_ref, graph_ref, acc_ref,
               *, heads, batch, anat):
    k = pl.program_id(0)
    a, denom = _attention_matrix(c_ref[...], asrc_ref[0], adst_ref[0])
    agg = jnp.dot(a.astype(BF), h_ref[0],
                  preferred_element_type=jnp.float32) * (1.0 / denom)

    @pl.when(k == 0)
    def _init():
        acc_ref[...] = agg

    @pl.when(k > 0)
    def _acc():
        acc_ref[...] += agg

    @pl.when(k == heads - 1)
    def _heads():
        h2 = acc_ref[...] * (1.0 / heads) + b2_ref[...]
        node_ref[...] = jnp.dot(h2, wn_ref[...],
                                preferred_element_type=jnp.float32) + bn_ref[...]
        row = lax.broadcasted_iota(jnp.int32, (batch, N), 0)
        col = lax.broadcasted_iota(jnp.int32, (batch, N), 1)
        pool = jnp.where((col >= row * anat) & (col < (row + 1) * anat),
                         1.0 / anat, 0.0)
        gmean = jnp.dot(pool, h2, preferred_element_type=jnp.float32)
        graph_ref[...] = (
            jnp.dot(gmean, wgt_ref[...], preferred_element_type=jnp.float32)
            + jnp.dot(gf_ref[...], wgb_ref[...],
                      preferred_element_type=jnp.float32)
            + bg_ref[...])


def _agg2(C, asrc_row, adst_col, h2pre, b2r, wn_p, bn_p, wgt_p, wgb_p, bg_p,
          gf, heads, och):
    batch = gf.shape[0]
    anat = N // batch
    return pl.pallas_call(
        functools.partial(_agg2_body, heads=heads, batch=batch, anat=anat),
        grid=(heads,),
        in_specs=[
            pl.BlockSpec((N, N), lambda k: (0, 0)),
            pl.BlockSpec((1, 1, N), lambda k: (k, 0, 0)),
            pl.BlockSpec((1, N, 1), lambda k: (k, 0, 0)),
            pl.BlockSpec((1, N, och), lambda k: (k, 0, 0)),
            pl.BlockSpec((1, och), lambda k: (0, 0)),
            pl.BlockSpec((och, 128), lambda k: (0, 0)),
            pl.BlockSpec((1, 128), lambda k: (0, 0)),
            pl.BlockSpec((och, 128), lambda k: (0, 0)),
            pl.BlockSpec((och, 128), lambda k: (0, 0)),
            pl.BlockSpec((1, 128), lambda k: (0, 0)),
            pl.BlockSpec((batch, och), lambda k: (0, 0)),
        ],
        out_specs=[
            pl.BlockSpec((N, 128), lambda k: (0, 0)),
            pl.BlockSpec((batch, 128), lambda k: (0, 0)),
        ],
        out_shape=[
            jax.ShapeDtypeStruct((N, 128), jnp.float32),
            jax.ShapeDtypeStruct((batch, 128), jnp.float32),
        ],
        scratch_shapes=[pltpu.VMEM((N, och), jnp.float32)],
    )(C, asrc_row, adst_col, h2pre, b2r, wn_p, bn_p, wgt_p, wgb_p, bg_p, gf)


def kernel(x, edge_index, global_feat, W1, as1, ad1, b1, W2, as2, ad2, b2,
           Wn, bn, Wg, bg):
    loops = jnp.arange(N, dtype=jnp.int32)
    n_fill = E_PAD - E_LOOPS
    src_pad = jnp.concatenate(
        [edge_index[0], loops, jnp.zeros((n_fill,), jnp.int32)])
    dst_pad = jnp.concatenate(
        [edge_index[1], loops, jnp.full((n_fill,), N, jnp.int32)])
    zeros_row = jnp.zeros((SUB_WORDS,), jnp.float32)

    C = _build_count(src_pad, dst_pad, zeros_row)

    h1 = _layer1(x, W1, as1[0][:, :, None], ad1[0][:, :, None], C,
                 b1.reshape(5, 1, 2048), heads=5, och=2048)

    h2pre, asrc2, adst2 = _mm2(h1, W2, as2.reshape(3, 1, 1024),
                               ad2.reshape(3, 1, 1024),
                               heads=3, och=1024, kblk=2048)

    wn_p = jnp.pad(Wn, ((0, 0), (0, 125)))
    bn_p = jnp.pad(bn, (0, 125)).reshape(1, 128)
    wgt_p = jnp.pad(Wg[:1024], ((0, 0), (0, 125)))
    wgb_p = jnp.pad(Wg[1024:], ((0, 0), (0, 125)))
    bg_p = jnp.pad(bg, (0, 125)).reshape(1, 128)

    node_p, graph_p = _agg2(C, asrc2.transpose(0, 2, 1), adst2, h2pre,
                            b2.reshape(1, 1024), wn_p, bn_p, wgt_p, wgb_p,
                            bg_p, global_feat, heads=3, och=1024)
    return node_p[:, :3], graph_p[:, :3]


# speedup vs baseline: 1.0609x; 1.0609x over previous
import functools

import jax
import jax.numpy as jnp
from jax import lax
from jax.experimental import pallas as pl
from jax.experimental.pallas import tpu as pltpu
from jax.experimental.pallas import tpu_sc as plsc

N = 1152
N2 = N * N
E_RAW = 19584
E_LOOPS = E_RAW + N
E_PAD = 24576
EDGES_PER_W = E_PAD // 16
CHUNKS_PER_W = EDGES_PER_W // 128
HALF = N // 2
HALF_WORDS = HALF * N
SUB_ROWS = HALF // 16
SUB_WORDS = SUB_ROWS * N
C_SH_LEN = HALF_WORDS + 128
BF = jnp.bfloat16


def _count_body(src_hbm, dst_hbm, zeros_hbm, out_hbm,
                src_v, dst_v, idx_v, val_v, c_sh, out_sem):
    cid = lax.axis_index("c")
    sid = lax.axis_index("s")
    lo = cid * HALF

    z = pltpu.async_copy(zeros_hbm, c_sh.at[pl.ds(sid * SUB_WORDS, SUB_WORDS)],
                         out_sem)
    base = pl.multiple_of(sid * EDGES_PER_W, EDGES_PER_W)
    pltpu.sync_copy(src_hbm.at[pl.ds(base, EDGES_PER_W)], src_v)
    pltpu.sync_copy(dst_hbm.at[pl.ds(base, EDGES_PER_W)], dst_v)

    for j in range(EDGES_PER_W // 16):
        d = dst_v[pl.ds(j * 16, 16)] - lo
        s = src_v[pl.ds(j * 16, 16)]
        flat = d * N + s
        ok = (d >= 0) & (d < HALF)
        dummy = HALF_WORDS + (j % 8) * 16 + lax.iota(jnp.int32, 16)
        idx_v[j // 8, pl.ds((j % 8) * 16, 16)] = jnp.where(ok, flat, dummy)
        val_v[j // 8, pl.ds((j % 8) * 16, 16)] = jnp.full((16,), 1.0,
                                                          jnp.float32)
    z.wait()
    plsc.subcore_barrier()

    for j in range(CHUNKS_PER_W):
        pltpu.sync_copy(val_v.at[j], c_sh.at[idx_v.at[j]], add=True)

    plsc.subcore_barrier()

    descs = []
    for r in range(SUB_ROWS):
        row = sid * SUB_ROWS + r
        descs.append(pltpu.async_copy(
            c_sh.at[pl.ds(row * N, N)], out_hbm.at[cid * HALF + row],
            out_sem))
    for d_ in descs:
        d_.wait()


def _build_count(src_pad, dst_pad, zeros_row):
    k = pl.kernel(
        _count_body,
        out_type=jax.ShapeDtypeStruct((N, N), jnp.float32),
        mesh=plsc.VectorSubcoreMesh(core_axis_name="c", subcore_axis_name="s"),
        scratch_types=[
            pltpu.VMEM((EDGES_PER_W,), jnp.int32),
            pltpu.VMEM((EDGES_PER_W,), jnp.int32),
            pltpu.VMEM((CHUNKS_PER_W, 128), jnp.int32),
            pltpu.VMEM((CHUNKS_PER_W, 128), jnp.float32),
            pltpu.VMEM_SHARED((C_SH_LEN,), jnp.float32),
            pltpu.SemaphoreType.DMA,
        ],
    )
    return k(src_pad, dst_pad, zeros_row)


def _attention_matrix(csum, asrc_row, adst_col):
    shift = jax.nn.leaky_relu(jnp.max(asrc_row) + jnp.max(adst_col), 0.2)
    logit = jax.nn.leaky_relu(asrc_row + adst_col, 0.2)
    a = csum * jnp.exp(logit - shift)
    denom = jnp.sum(a, axis=1, keepdims=True) + 1e-16
    return a, denom


def _l1_body(x_ref, w_ref, as_ref, ad_ref, c_ref, b_ref, out_ref):
    w1 = w_ref[...]
    x = x_ref[...]
    wv_s = jnp.dot(w1, as_ref[0], preferred_element_type=jnp.float32)
    wv_d = jnp.dot(w1, ad_ref[0], preferred_element_type=jnp.float32)
    asrc_row = lax.dot_general(wv_s, x, (((0,), (1,)), ((), ())),
                               preferred_element_type=jnp.float32)
    adst_col = jnp.dot(x, wv_d, preferred_element_type=jnp.float32)
    a, denom = _attention_matrix(c_ref[...], asrc_row, adst_col)
    ax = jnp.dot(a.astype(BF), x.astype(BF),
                 preferred_element_type=jnp.float32) * (1.0 / denom)
    h1k = jnp.dot(ax.astype(BF), w1.astype(BF),
                  preferred_element_type=jnp.float32) + b_ref[0]
    out_ref[...] = jax.nn.leaky_relu(h1k, 0.01).astype(BF)


def _layer1(x, W1, as1c, ad1c, C, b1r, heads, och):
    d_in = x.shape[1]
    return pl.pallas_call(
        _l1_body,
        grid=(heads,),
        in_specs=[
            pl.BlockSpec((N, d_in), lambda k: (0, 0)),
            pl.BlockSpec((d_in, och), lambda k: (0, k)),
            pl.BlockSpec((1, och, 1), lambda k: (k, 0, 0)),
            pl.BlockSpec((1, och, 1), lambda k: (k, 0, 0)),
            pl.BlockSpec((N, N), lambda k: (0, 0)),
            pl.BlockSpec((1, 1, och), lambda k: (k, 0, 0)),
        ],
        out_specs=pl.BlockSpec((N, och), lambda k: (0, k)),
        out_shape=jax.ShapeDtypeStruct((N, heads * och), BF),
    )(x, W1, as1c, ad1c, C, b1r)


def _mm2_body(h1_ref, w_ref, as_ref, ad_ref, out_ref, asrc_ref, adst_ref,
              acc_ref, *, kk_steps):
    kk = pl.program_id(0)
    kh = pl.program_id(1)
    part = jnp.dot(h1_ref[...], w_ref[...].astype(BF),
                   preferred_element_type=jnp.float32)

    @pl.when(kk == 0)
    def _init():
        acc_ref[kh] = part

    @pl.when(kk > 0)
    def _acc():
        acc_ref[kh] += part

    @pl.when(kk == kk_steps - 1)
    def _finish():
        hk = acc_ref[kh]
        out_ref[kh] = hk.astype(BF)
        asrc_ref[kh] = jnp.sum(hk * as_ref[kh], axis=1, keepdims=True)
        adst_ref[kh] = jnp.sum(hk * ad_ref[kh], axis=1, keepdims=True)


def _mm2(h1, W2, as2r, ad2r, heads, och, kblk):
    kin = h1.shape[1]
    kk_steps = kin // kblk
    return pl.pallas_call(
        functools.partial(_mm2_body, kk_steps=kk_steps),
        grid=(kk_steps, heads),
        in_specs=[
            pl.BlockSpec((N, kblk), lambda kk, kh: (0, kk)),
            pl.BlockSpec((kblk, och), lambda kk, kh: (kk, kh)),
            pl.BlockSpec((heads, 1, och), lambda kk, kh: (0, 0, 0)),
            pl.BlockSpec((heads, 1, och), lambda kk, kh: (0, 0, 0)),
        ],
        out_specs=[
            pl.BlockSpec((heads, N, och), lambda kk, kh: (0, 0, 0)),
            pl.BlockSpec((heads, N, 1), lambda kk, kh: (0, 0, 0)),
            pl.BlockSpec((heads, N, 1), lambda kk, kh: (0, 0, 0)),
        ],
        out_shape=[
            jax.ShapeDtypeStruct((heads, N, och), BF),
            jax.ShapeDtypeStruct((heads, N, 1), jnp.float32),
            jax.ShapeDtypeStruct((heads, N, 1), jnp.float32),
        ],
        scratch_shapes=[pltpu.VMEM((heads, N, och), jnp.float32)],
    )(h1, W2, as2r, ad2r)


def _agg2_body(c_ref, asrc_ref, adst_ref, h_ref, b2_ref, wn_ref, bn_ref,
               wgt_ref, wgb_ref, bg_ref, gf_ref, node_ref, graph_ref, acc_ref,
               *, heads, batch, anat):
    k = pl.program_id(0)
    a, denom = _attention_matrix(c_ref[...], asrc_ref[0], adst_ref[0])
    agg = jnp.dot(a.astype(BF), h_ref[0],
                  preferred_element_type=jnp.float32) * (1.0 / denom)

    @pl.when(k == 0)
    def _init():
        acc_ref[...] = agg

    @pl.when(k > 0)
    def _acc():
        acc_ref[...] += agg

    @pl.when(k == heads - 1)
    def _heads():
        h2 = acc_ref[...] * (1.0 / heads) + b2_ref[...]
        node_ref[...] = jnp.dot(h2, wn_ref[...],
                                preferred_element_type=jnp.float32) + bn_ref[...]
        row = lax.broadcasted_iota(jnp.int32, (batch, N), 0)
        col = lax.broadcasted_iota(jnp.int32, (batch, N), 1)
        pool = jnp.where((col >= row * anat) & (col < (row + 1) * anat),
                         1.0 / anat, 0.0)
        gmean = jnp.dot(pool, h2, preferred_element_type=jnp.float32)
        graph_ref[...] = (
            jnp.dot(gmean, wgt_ref[...], preferred_element_type=jnp.float32)
            + jnp.dot(gf_ref[...], wgb_ref[...],
                      preferred_element_type=jnp.float32)
            + bg_ref[...])


def _agg2(C, asrc_row, adst_col, h2pre, b2r, wn_p, bn_p, wgt_p, wgb_p, bg_p,
          gf, heads, och):
    batch = gf.shape[0]
    anat = N // batch
    return pl.pallas_call(
        functools.partial(_agg2_body, heads=heads, batch=batch, anat=anat),
        grid=(heads,),
        in_specs=[
            pl.BlockSpec((N, N), lambda k: (0, 0)),
            pl.BlockSpec((1, 1, N), lambda k: (k, 0, 0)),
            pl.BlockSpec((1, N, 1), lambda k: (k, 0, 0)),
            pl.BlockSpec((1, N, och), lambda k: (k, 0, 0)),
            pl.BlockSpec((1, och), lambda k: (0, 0)),
            pl.BlockSpec((och, 128), lambda k: (0, 0)),
            pl.BlockSpec((1, 128), lambda k: (0, 0)),
            pl.BlockSpec((och, 128), lambda k: (0, 0)),
            pl.BlockSpec((och, 128), lambda k: (0, 0)),
            pl.BlockSpec((1, 128), lambda k: (0, 0)),
            pl.BlockSpec((batch, och), lambda k: (0, 0)),
        ],
        out_specs=[
            pl.BlockSpec((N, 128), lambda k: (0, 0)),
            pl.BlockSpec((batch, 128), lambda k: (0, 0)),
        ],
        out_shape=[
            jax.ShapeDtypeStruct((N, 128), jnp.float32),
            jax.ShapeDtypeStruct((batch, 128), jnp.float32),
        ],
        scratch_shapes=[pltpu.VMEM((N, och), jnp.float32)],
    )(C, asrc_row, adst_col, h2pre, b2r, wn_p, bn_p, wgt_p, wgb_p, bg_p, gf)


def kernel(x, edge_index, global_feat, W1, as1, ad1, b1, W2, as2, ad2, b2,
           Wn, bn, Wg, bg):
    loops = jnp.arange(N, dtype=jnp.int32)
    n_fill = E_PAD - E_LOOPS
    src_pad = jnp.concatenate(
        [edge_index[0], loops, jnp.zeros((n_fill,), jnp.int32)])
    dst_pad = jnp.concatenate(
        [edge_index[1], loops, jnp.full((n_fill,), N, jnp.int32)])
    zeros_row = jnp.zeros((SUB_WORDS,), jnp.float32)

    C = _build_count(src_pad, dst_pad, zeros_row)

    h1 = _layer1(x, W1, as1[0][:, :, None], ad1[0][:, :, None], C,
                 b1.reshape(5, 1, 2048), heads=5, och=2048)

    h2pre, asrc2, adst2 = _mm2(h1, W2, as2.reshape(3, 1, 1024),
                               ad2.reshape(3, 1, 1024),
                               heads=3, och=1024, kblk=2048)

    wn_p = jnp.pad(Wn, ((0, 0), (0, 125)))
    bn_p = jnp.pad(bn, (0, 125)).reshape(1, 128)
    wgt_p = jnp.pad(Wg[:1024], ((0, 0), (0, 125)))
    wgb_p = jnp.pad(Wg[1024:], ((0, 0), (0, 125)))
    bg_p = jnp.pad(bg, (0, 125)).reshape(1, 128)

    node_p, graph_p = _agg2(C, asrc2.transpose(0, 2, 1), adst2, h2pre,
                            b2.reshape(1, 1024), wn_p, bn_p, wgt_p, wgb_p,
                            bg_p, global_feat, heads=3, och=1024)
    return node_p[:, :3], graph_p[:, :3]
